# Initial kernel scaffold; baseline (speedup 1.0000x reference)
import functools

import jax
import jax.numpy as jnp
from jax import lax
from jax.experimental import pallas as pl
from jax.experimental.pallas import tpu as pltpu
from jax.experimental.pallas import tpu_sc as plsc

N = 10000
E = 320000
NC = 2
NS = 16
NW = NC * NS
EPW = E // NW
C = 80
NCHUNK = EPW // C
ROWS_PT = N // NS
DEGW = 8


def _sc_aggregate(g, src, dst, d):
    mesh = plsc.VectorSubcoreMesh(core_axis_name="c", subcore_axis_name="s")

    @functools.partial(
        pl.kernel,
        out_type=(
            jax.ShapeDtypeStruct((NC, N, d), jnp.float32),
            jax.ShapeDtypeStruct((NC, N, DEGW), jnp.float32),
        ),
        mesh=mesh,
        scratch_types=[
            pltpu.VMEM((C,), jnp.int32),
            pltpu.VMEM((C,), jnp.int32),
            pltpu.VMEM((C, d), jnp.float32),
            pltpu.VMEM((C, DEGW), jnp.float32),
            pltpu.VMEM_SHARED((N, d), jnp.float32),
            pltpu.VMEM_SHARED((N, DEGW), jnp.float32),
            pltpu.SemaphoreType.DMA,
        ],
    )
    def agg(g_hbm, src_hbm, dst_hbm, zrow_hbm, zdeg_hbm, ones_hbm,
            sum_hbm, deg_hbm, sidx, didx, rows, ones_v, acc_sh, deg_sh, sem):
        ci = lax.axis_index("c")
        si = lax.axis_index("s")
        rbase = si * ROWS_PT
        pltpu.sync_copy(zrow_hbm, acc_sh.at[pl.ds(rbase, ROWS_PT), :])
        pltpu.sync_copy(zdeg_hbm, deg_sh.at[pl.ds(rbase, ROWS_PT), :])
        pltpu.sync_copy(ones_hbm, ones_v)
        plsc.subcore_barrier()

        w = si * NC + ci
        ebase = w * EPW

        def body(i, carry):
            off = ebase + i * C
            pltpu.sync_copy(src_hbm.at[pl.ds(off, C)], sidx)
            pltpu.sync_copy(dst_hbm.at[pl.ds(off, C)], didx)
            pltpu.async_copy(g_hbm.at[sidx], rows, sem).wait()
            pltpu.sync_copy(rows, acc_sh.at[didx], add=True)
            pltpu.sync_copy(ones_v, deg_sh.at[didx], add=True)
            return carry

        lax.fori_loop(0, NCHUNK, body, 0)
        plsc.subcore_barrier()
        pltpu.sync_copy(acc_sh.at[pl.ds(rbase, ROWS_PT), :],
                        sum_hbm.at[ci, pl.ds(rbase, ROWS_PT), :])
        pltpu.sync_copy(deg_sh.at[pl.ds(rbase, ROWS_PT), :],
                        deg_hbm.at[ci, pl.ds(rbase, ROWS_PT), :])

    zrow = jnp.zeros((ROWS_PT, d), jnp.float32)
    zdeg = jnp.zeros((ROWS_PT, DEGW), jnp.float32)
    ones = jnp.ones((C, DEGW), jnp.float32)
    return agg(g, src, dst, zrow, zdeg, ones)


_BN = 2500


def _tc_first(h, w_neigh, w_self, b):
    d_out = w_neigh.shape[1]

    def body(h_ref, wn_ref, ws_ref, b_ref, g_ref, s_ref):
        h_blk = h_ref[...]
        g_ref[...] = jnp.dot(h_blk, wn_ref[...],
                             preferred_element_type=jnp.float32)
        s_ref[...] = jnp.dot(h_blk, ws_ref[...],
                             preferred_element_type=jnp.float32) + b_ref[...]

    return pl.pallas_call(
        body,
        grid=(N // _BN,),
        in_specs=[
            pl.BlockSpec((_BN, h.shape[1]), lambda i: (i, 0)),
            pl.BlockSpec(w_neigh.shape, lambda i: (0, 0)),
            pl.BlockSpec(w_self.shape, lambda i: (0, 0)),
            pl.BlockSpec((1, d_out), lambda i: (0, 0)),
        ],
        out_specs=[
            pl.BlockSpec((_BN, d_out), lambda i: (i, 0)),
            pl.BlockSpec((_BN, d_out), lambda i: (i, 0)),
        ],
        out_shape=[
            jax.ShapeDtypeStruct((N, d_out), jnp.float32),
            jax.ShapeDtypeStruct((N, d_out), jnp.float32),
        ],
    )(h, w_neigh, w_self, b.reshape(1, -1))


def _tc_mid(s_prev, ssum, deg, w_neigh, w_self, b):
    d_in = s_prev.shape[1]
    d_out = w_neigh.shape[1]

    def body(sp_ref, sum_ref, deg_ref, wn_ref, ws_ref, b_ref, g_ref, s_ref):
        total = sum_ref[0] + sum_ref[1]
        degc = deg_ref[0, :, 0:1] + deg_ref[1, :, 0:1]
        h_blk = jnp.maximum(
            sp_ref[...] + total / jnp.maximum(degc, 1.0), 0.0)
        g_ref[...] = jnp.dot(h_blk, wn_ref[...],
                             preferred_element_type=jnp.float32)
        s_ref[...] = jnp.dot(h_blk, ws_ref[...],
                             preferred_element_type=jnp.float32) + b_ref[...]

    return pl.pallas_call(
        body,
        grid=(N // _BN,),
        in_specs=[
            pl.BlockSpec((_BN, d_in), lambda i: (i, 0)),
            pl.BlockSpec((NC, _BN, d_in), lambda i: (0, i, 0)),
            pl.BlockSpec((NC, _BN, DEGW), lambda i: (0, i, 0)),
            pl.BlockSpec(w_neigh.shape, lambda i: (0, 0)),
            pl.BlockSpec(w_self.shape, lambda i: (0, 0)),
            pl.BlockSpec((1, d_out), lambda i: (0, 0)),
        ],
        out_specs=[
            pl.BlockSpec((_BN, d_out), lambda i: (i, 0)),
            pl.BlockSpec((_BN, d_out), lambda i: (i, 0)),
        ],
        out_shape=[
            jax.ShapeDtypeStruct((N, d_out), jnp.float32),
            jax.ShapeDtypeStruct((N, d_out), jnp.float32),
        ],
    )(s_prev, ssum, deg, w_neigh, w_self, b.reshape(1, -1))


def _tc_last(s_prev, ssum, deg):
    d = s_prev.shape[1]

    def body(sp_ref, sum_ref, deg_ref, o_ref):
        total = sum_ref[0] + sum_ref[1]
        degc = deg_ref[0, :, 0:1] + deg_ref[1, :, 0:1]
        o_ref[...] = sp_ref[...] + total / jnp.maximum(degc, 1.0)

    return pl.pallas_call(
        body,
        grid=(N // _BN,),
        in_specs=[
            pl.BlockSpec((_BN, d), lambda i: (i, 0)),
            pl.BlockSpec((NC, _BN, d), lambda i: (0, i, 0)),
            pl.BlockSpec((NC, _BN, DEGW), lambda i: (0, i, 0)),
        ],
        out_specs=pl.BlockSpec((_BN, d), lambda i: (i, 0)),
        out_shape=jax.ShapeDtypeStruct((N, d), jnp.float32),
    )(s_prev, ssum, deg)


def kernel(inputs, edge_index0, edge_index1, edge_index2,
           W_self0, W_neigh0, b0, W_self1, W_neigh1, b1,
           W_self2, W_neigh2, b2):
    g0, s0 = _tc_first(inputs, W_neigh0, W_self0, b0)
    sum0, deg0 = _sc_aggregate(g0, edge_index0[0], edge_index0[1], 128)
    g1, s1 = _tc_mid(s0, sum0, deg0, W_neigh1, W_self1, b1)
    sum1, deg1 = _sc_aggregate(g1, edge_index1[0], edge_index1[1], 128)
    g2, s2 = _tc_mid(s1, sum1, deg1, W_neigh2, W_self2, b2)
    sum2, deg2 = _sc_aggregate(g2, edge_index2[0], edge_index2[1], 64)
    return _tc_last(s2, sum2, deg2)


# speedup vs baseline: 3.8390x; 3.8390x over previous
import functools

import jax
import jax.numpy as jnp
from jax import lax
from jax.experimental import pallas as pl
from jax.experimental.pallas import tpu as pltpu
from jax.experimental.pallas import tpu_sc as plsc

N = 10000
E = 320000
NC = 2
NS = 16
NW = NC * NS
EPW = E // NW
C = 80
NCHUNK = EPW // C
RPT = 640
DEGW = 128


def _mesh():
    return plsc.VectorSubcoreMesh(core_axis_name="c", subcore_axis_name="s",
                                  num_cores=NC, num_subcores=NS)


def _row_chunks(si):
    return jnp.where(si == NS - 1, (N - (NS - 1) * RPT) // C, RPT // C)


def _sc_sum(g, src, dst):
    d = g.shape[1]

    @functools.partial(
        pl.kernel,
        out_type=jax.ShapeDtypeStruct((NC, N, d), jnp.float32),
        mesh=_mesh(),
        scratch_types=[
            pltpu.VMEM((C,), jnp.int32),
            pltpu.VMEM((C,), jnp.int32),
            pltpu.VMEM((C, d), jnp.float32),
            pltpu.VMEM_SHARED((N, d), jnp.float32),
            pltpu.SemaphoreType.DMA,
        ],
    )
    def agg(g_hbm, src_hbm, dst_hbm, zrow_hbm, sum_hbm,
            sidx, didx, rows, acc_sh, sem):
        ci = lax.axis_index("c")
        si = lax.axis_index("s")
        rbase = si * RPT
        nrch = _row_chunks(si)

        pltpu.sync_copy(zrow_hbm, rows)

        def zbody(j, carry):
            pltpu.sync_copy(rows, acc_sh.at[pl.ds(rbase + j * C, C), :])
            return carry

        lax.fori_loop(0, nrch, zbody, 0)
        plsc.subcore_barrier()

        ebase = (si * NC + ci) * EPW

        def body(i, carry):
            off = ebase + i * C
            pltpu.sync_copy(src_hbm.at[pl.ds(off, C)], sidx)
            pltpu.sync_copy(dst_hbm.at[pl.ds(off, C)], didx)
            pltpu.async_copy(g_hbm.at[sidx], rows, sem).wait()
            pltpu.sync_copy(rows, acc_sh.at[didx], add=True)
            return carry

        lax.fori_loop(0, NCHUNK, body, 0)
        plsc.subcore_barrier()

        def pbody(j, carry):
            off = rbase + j * C
            pltpu.sync_copy(acc_sh.at[pl.ds(off, C), :], rows)
            pltpu.sync_copy(rows, sum_hbm.at[ci, pl.ds(off, C), :])
            return carry

        lax.fori_loop(0, nrch, pbody, 0)

    zrow = jnp.zeros((C, d), jnp.float32)
    return agg(g, src, dst, zrow)


def _sc_deg(dst):
    @functools.partial(
        pl.kernel,
        out_type=jax.ShapeDtypeStruct((NC, N, DEGW), jnp.float32),
        mesh=_mesh(),
        scratch_types=[
            pltpu.VMEM((C,), jnp.int32),
            pltpu.VMEM((C, DEGW), jnp.float32),
            pltpu.VMEM_SHARED((N, DEGW), jnp.float32),
            pltpu.SemaphoreType.DMA,
        ],
    )
    def deg(dst_hbm, zdeg_hbm, ones_hbm, deg_hbm, didx, buf, deg_sh, sem):
        ci = lax.axis_index("c")
        si = lax.axis_index("s")
        rbase = si * RPT
        nrch = _row_chunks(si)

        pltpu.sync_copy(zdeg_hbm, buf)

        def zbody(j, carry):
            pltpu.sync_copy(buf, deg_sh.at[pl.ds(rbase + j * C, C), :])
            return carry

        lax.fori_loop(0, nrch, zbody, 0)
        pltpu.sync_copy(ones_hbm, buf)
        plsc.subcore_barrier()

        ebase = (si * NC + ci) * EPW

        def body(i, carry):
            off = ebase + i * C
            pltpu.sync_copy(dst_hbm.at[pl.ds(off, C)], didx)
            pltpu.sync_copy(buf, deg_sh.at[didx], add=True)
            return carry

        lax.fori_loop(0, NCHUNK, body, 0)
        plsc.subcore_barrier()

        def pbody(j, carry):
            off = rbase + j * C
            pltpu.sync_copy(deg_sh.at[pl.ds(off, C), :], buf)
            pltpu.sync_copy(buf, deg_hbm.at[ci, pl.ds(off, C), :])
            return carry

        lax.fori_loop(0, nrch, pbody, 0)

    zdeg = jnp.zeros((C, DEGW), jnp.float32)
    ones = jnp.ones((C, DEGW), jnp.float32)
    return deg(dst, zdeg, ones)


_BN = 2000


def _tc_first(h, w_neigh, w_self, b):
    d_out = w_neigh.shape[1]

    def body(h_ref, wn_ref, ws_ref, b_ref, g_ref, s_ref):
        h_blk = h_ref[...]
        g_ref[...] = jnp.dot(h_blk, wn_ref[...],
                             preferred_element_type=jnp.float32)
        s_ref[...] = jnp.dot(h_blk, ws_ref[...],
                             preferred_element_type=jnp.float32) + b_ref[...]

    return pl.pallas_call(
        body,
        grid=(N // _BN,),
        in_specs=[
            pl.BlockSpec((_BN, h.shape[1]), lambda i: (i, 0)),
            pl.BlockSpec(w_neigh.shape, lambda i: (0, 0)),
            pl.BlockSpec(w_self.shape, lambda i: (0, 0)),
            pl.BlockSpec((1, d_out), lambda i: (0, 0)),
        ],
        out_specs=[
            pl.BlockSpec((_BN, d_out), lambda i: (i, 0)),
            pl.BlockSpec((_BN, d_out), lambda i: (i, 0)),
        ],
        out_shape=[
            jax.ShapeDtypeStruct((N, d_out), jnp.float32),
            jax.ShapeDtypeStruct((N, d_out), jnp.float32),
        ],
    )(h, w_neigh, w_self, b.reshape(1, -1))


def _tc_mid(s_prev, ssum, deg, w_neigh, w_self, b):
    d_in = s_prev.shape[1]
    d_out = w_neigh.shape[1]

    def body(sp_ref, sum_ref, deg_ref, wn_ref, ws_ref, b_ref, g_ref, s_ref):
        total = sum_ref[0] + sum_ref[1]
        degc = deg_ref[0, :, 0:1] + deg_ref[1, :, 0:1]
        h_blk = jnp.maximum(
            sp_ref[...] + total / jnp.maximum(degc, 1.0), 0.0)
        g_ref[...] = jnp.dot(h_blk, wn_ref[...],
                             preferred_element_type=jnp.float32)
        s_ref[...] = jnp.dot(h_blk, ws_ref[...],
                             preferred_element_type=jnp.float32) + b_ref[...]

    return pl.pallas_call(
        body,
        grid=(N // _BN,),
        in_specs=[
            pl.BlockSpec((_BN, d_in), lambda i: (i, 0)),
            pl.BlockSpec((NC, _BN, d_in), lambda i: (0, i, 0)),
            pl.BlockSpec((NC, _BN, DEGW), lambda i: (0, i, 0)),
            pl.BlockSpec(w_neigh.shape, lambda i: (0, 0)),
            pl.BlockSpec(w_self.shape, lambda i: (0, 0)),
            pl.BlockSpec((1, d_out), lambda i: (0, 0)),
        ],
        out_specs=[
            pl.BlockSpec((_BN, d_out), lambda i: (i, 0)),
            pl.BlockSpec((_BN, d_out), lambda i: (i, 0)),
        ],
        out_shape=[
            jax.ShapeDtypeStruct((N, d_out), jnp.float32),
            jax.ShapeDtypeStruct((N, d_out), jnp.float32),
        ],
    )(s_prev, ssum, deg, w_neigh, w_self, b.reshape(1, -1))


def _tc_mid2(s_prev, ssum, deg, w_self, b):
    d_in = s_prev.shape[1]
    d_out = w_self.shape[1]

    def body(sp_ref, sum_ref, deg_ref, ws_ref, b_ref, h_ref, s_ref):
        total = sum_ref[0] + sum_ref[1]
        degc = deg_ref[0, :, 0:1] + deg_ref[1, :, 0:1]
        h_blk = jnp.maximum(
            sp_ref[...] + total / jnp.maximum(degc, 1.0), 0.0)
        h_ref[...] = h_blk
        s_ref[...] = jnp.dot(h_blk, ws_ref[...],
                             preferred_element_type=jnp.float32) + b_ref[...]

    return pl.pallas_call(
        body,
        grid=(N // _BN,),
        in_specs=[
            pl.BlockSpec((_BN, d_in), lambda i: (i, 0)),
            pl.BlockSpec((NC, _BN, d_in), lambda i: (0, i, 0)),
            pl.BlockSpec((NC, _BN, DEGW), lambda i: (0, i, 0)),
            pl.BlockSpec(w_self.shape, lambda i: (0, 0)),
            pl.BlockSpec((1, d_out), lambda i: (0, 0)),
        ],
        out_specs=[
            pl.BlockSpec((_BN, d_in), lambda i: (i, 0)),
            pl.BlockSpec((_BN, d_out), lambda i: (i, 0)),
        ],
        out_shape=[
            jax.ShapeDtypeStruct((N, d_in), jnp.float32),
            jax.ShapeDtypeStruct((N, d_out), jnp.float32),
        ],
    )(s_prev, ssum, deg, w_self, b.reshape(1, -1))


def _tc_last(s_prev, ssum, deg, w_neigh):
    d_in = w_neigh.shape[0]
    d_out = w_neigh.shape[1]

    def body(sp_ref, sum_ref, deg_ref, wn_ref, o_ref):
        total = sum_ref[0] + sum_ref[1]
        degc = deg_ref[0, :, 0:1] + deg_ref[1, :, 0:1]
        h_neigh = total / jnp.maximum(degc, 1.0)
        o_ref[...] = sp_ref[...] + jnp.dot(
            h_neigh, wn_ref[...], preferred_element_type=jnp.float32)

    return pl.pallas_call(
        body,
        grid=(N // _BN,),
        in_specs=[
            pl.BlockSpec((_BN, d_out), lambda i: (i, 0)),
            pl.BlockSpec((NC, _BN, d_in), lambda i: (0, i, 0)),
            pl.BlockSpec((NC, _BN, DEGW), lambda i: (0, i, 0)),
            pl.BlockSpec(w_neigh.shape, lambda i: (0, 0)),
        ],
        out_specs=pl.BlockSpec((_BN, d_out), lambda i: (i, 0)),
        out_shape=jax.ShapeDtypeStruct((N, d_out), jnp.float32),
    )(s_prev, ssum, deg, w_neigh)


def kernel(inputs, edge_index0, edge_index1, edge_index2,
           W_self0, W_neigh0, b0, W_self1, W_neigh1, b1,
           W_self2, W_neigh2, b2):
    g0, s0 = _tc_first(inputs, W_neigh0, W_self0, b0)
    sum0 = _sc_sum(g0, edge_index0[0], edge_index0[1])
    deg0 = _sc_deg(edge_index0[1])
    g1, s1 = _tc_mid(s0, sum0, deg0, W_neigh1, W_self1, b1)
    sum1 = _sc_sum(g1, edge_index1[0], edge_index1[1])
    deg1 = _sc_deg(edge_index1[1])
    h2, s2 = _tc_mid2(s1, sum1, deg1, W_self2, b2)
    sum2 = _sc_sum(h2, edge_index2[0], edge_index2[1])
    deg2 = _sc_deg(edge_index2[1])
    return _tc_last(s2, sum2, deg2, W_neigh2)
